# Initial kernel scaffold; baseline (speedup 1.0000x reference)
import jax
import jax.numpy as jnp
from jax.experimental import pallas as pl


def kernel(x):
    raise NotImplementedError("write your pallas kernel here")


# speedup vs baseline: 4.7800x; 4.7800x over previous
import jax
import jax.numpy as jnp
from jax import lax
from jax.experimental import pallas as pl

_K = 64
_BLOCK_B = 8


def _body(x_ref, o_ref):
    xv = x_ref[...]
    i = lax.bitcast_convert_type(xv, jnp.int32)
    key = i ^ (lax.shift_right_arithmetic(i, 31) & jnp.int32(0x7FFFFFFF))

    sign = jnp.int32(-2147483648)

    def step(t, uprefix):
        bit = jnp.int32(1) << (jnp.int32(31) - t)
        ucand = uprefix | bit
        cnt = jnp.sum((key >= (ucand ^ sign)).astype(jnp.int32),
                      axis=1, keepdims=True)
        return jnp.where(cnt >= _K, ucand, uprefix)

    init = jnp.zeros((xv.shape[0], 1), jnp.int32)
    uthresh = lax.fori_loop(0, 32, step, init)
    thresh = uthresh ^ sign

    o_ref[...] = jnp.where(key >= thresh, jnp.maximum(xv, 0.0), 0.0)


def kernel(x):
    B, N = x.shape
    grid = (B // _BLOCK_B,)
    return pl.pallas_call(
        _body,
        grid=grid,
        in_specs=[pl.BlockSpec((_BLOCK_B, N), lambda b: (b, 0))],
        out_specs=pl.BlockSpec((_BLOCK_B, N), lambda b: (b, 0)),
        out_shape=jax.ShapeDtypeStruct((B, N), x.dtype),
    )(x)


# speedup vs baseline: 6.6178x; 1.3845x over previous
import jax
import jax.numpy as jnp
from jax import lax
from jax.experimental import pallas as pl

_K = 64
_BLOCK_B = 8


def _body(x_ref, o_ref):
    xv = x_ref[...]
    i = lax.bitcast_convert_type(xv, jnp.int32)
    key = i ^ (lax.shift_right_arithmetic(i, 31) & jnp.int32(0x7FFFFFFF))

    sign = jnp.int32(-2147483648)

    def cond(state):
        t, _, cur = state
        return jnp.logical_and(t < 32, jnp.any(cur != _K))

    def step(state):
        t, uprefix, cur = state
        bit = jnp.int32(1) << (jnp.int32(31) - t)
        ucand = uprefix | bit
        cnt = jnp.sum((key >= (ucand ^ sign)).astype(jnp.int32),
                      axis=1, keepdims=True)
        take = cnt >= _K
        return (t + 1,
                jnp.where(take, ucand, uprefix),
                jnp.where(take, cnt, cur))

    BB = xv.shape[0]
    init = (jnp.int32(0),
            jnp.zeros((BB, 1), jnp.int32),
            jnp.full((BB, 1), jnp.int32(xv.shape[1])))
    _, uthresh, _ = lax.while_loop(cond, step, init)
    thresh = uthresh ^ sign

    o_ref[...] = jnp.where(key >= thresh, jnp.maximum(xv, 0.0), 0.0)


def kernel(x):
    B, N = x.shape
    grid = (B // _BLOCK_B,)
    return pl.pallas_call(
        _body,
        grid=grid,
        in_specs=[pl.BlockSpec((_BLOCK_B, N), lambda b: (b, 0))],
        out_specs=pl.BlockSpec((_BLOCK_B, N), lambda b: (b, 0)),
        out_shape=jax.ShapeDtypeStruct((B, N), x.dtype),
    )(x)
